# baseline (speedup 1.0000x reference)
import functools

import jax
import jax.numpy as jnp
from jax import lax
from jax.experimental import pallas as pl
from jax.experimental.pallas import tpu as pltpu
from jax.experimental.pallas import tpu_sc as plsc

EMBED_DIM = 32
_SEQ = 50
_NSAMPLES = 16384
_NC, _NS = 2, 16
_NW = _NC * _NS
_S_PER_W = _NSAMPLES // _NW
_K = 16
_NBLK = _S_PER_W // _K

_mesh = plsc.VectorSubcoreMesh(core_axis_name="c", subcore_axis_name="s")


@functools.partial(
    pl.kernel,
    out_type=jax.ShapeDtypeStruct((_NSAMPLES, _SEQ, EMBED_DIM), jnp.float32),
    mesh=_mesh,
    compiler_params=pltpu.CompilerParams(use_tc_tiling_on_sc=False),
    scratch_types=[
        pltpu.VMEM((_S_PER_W, _SEQ), jnp.int32),
        pltpu.VMEM((2, _K, _SEQ, EMBED_DIM), jnp.float32),
        pltpu.SemaphoreType.DMA,
        pltpu.SemaphoreType.DMA,
    ],
)
def _gather(idx_hbm, table_hbm, out_hbm, idx_v, rows_v, sem0, sem1):
    wid = lax.axis_index("s") * _NC + lax.axis_index("c")
    sample0 = wid * _S_PER_W
    pltpu.sync_copy(idx_hbm.at[pl.ds(sample0, _S_PER_W)], idx_v)

    def start(blk, buf, sem):
        for j in range(_K):
            pltpu.async_copy(
                table_hbm.at[idx_v.at[blk * _K + j]], rows_v.at[buf, j], sem
            )

    def wait(buf, sem):
        for j in range(_K):
            pltpu.make_async_copy(
                table_hbm.at[idx_v.at[0]], rows_v.at[buf, j], sem
            ).wait()

    def flush(blk, buf):
        pltpu.sync_copy(
            rows_v.at[buf], out_hbm.at[pl.ds(sample0 + blk * _K, _K)]
        )

    start(0, 0, sem0)

    def body(i, carry):
        blk = 2 * i
        wait(0, sem0)
        start(blk + 1, 1, sem1)
        flush(blk, 0)
        wait(1, sem1)

        @pl.when(blk + 2 < _NBLK)
        def _start_next():
            start(blk + 2, 0, sem0)

        flush(blk + 1, 1)
        return carry

    lax.fori_loop(0, _NBLK // 2, body, 0)


def kernel(idx, embedding):
    return _gather(idx.astype(jnp.int32), embedding)


# speedup vs baseline: 1.0068x; 1.0068x over previous
import functools

import jax
import jax.numpy as jnp
from jax import lax
from jax.experimental import pallas as pl
from jax.experimental.pallas import tpu as pltpu
from jax.experimental.pallas import tpu_sc as plsc

EMBED_DIM = 32
_SEQ = 50
_NSAMPLES = 16384
_NC, _NS = 2, 16
_NW = _NC * _NS
_S_PER_W = _NSAMPLES // _NW
_K = 32
_NBLK = _S_PER_W // _K

_mesh = plsc.VectorSubcoreMesh(core_axis_name="c", subcore_axis_name="s")


@functools.partial(
    pl.kernel,
    out_type=jax.ShapeDtypeStruct((_NSAMPLES, _SEQ, EMBED_DIM), jnp.float32),
    mesh=_mesh,
    compiler_params=pltpu.CompilerParams(use_tc_tiling_on_sc=False),
    scratch_types=[
        pltpu.VMEM((_S_PER_W, _SEQ), jnp.int32),
        pltpu.VMEM((2, _K, _SEQ, EMBED_DIM), jnp.float32),
        pltpu.SemaphoreType.DMA,
        pltpu.SemaphoreType.DMA,
    ],
)
def _gather(idx_hbm, table_hbm, out_hbm, idx_v, rows_v, sem0, sem1):
    wid = lax.axis_index("s") * _NC + lax.axis_index("c")
    sample0 = wid * _S_PER_W
    pltpu.sync_copy(idx_hbm.at[pl.ds(sample0, _S_PER_W)], idx_v)

    def start(blk, buf, sem):
        for j in range(_K):
            pltpu.async_copy(
                table_hbm.at[idx_v.at[blk * _K + j]], rows_v.at[buf, j], sem
            )

    def wait(buf, sem):
        pltpu.make_async_copy(
            out_hbm.at[pl.ds(sample0, _K)], rows_v.at[buf], sem
        ).wait()

    def flush(blk, buf):
        pltpu.sync_copy(
            rows_v.at[buf], out_hbm.at[pl.ds(sample0 + blk * _K, _K)]
        )

    start(0, 0, sem0)

    def body(i, carry):
        blk = 2 * i
        wait(0, sem0)
        start(blk + 1, 1, sem1)
        flush(blk, 0)
        wait(1, sem1)

        @pl.when(blk + 2 < _NBLK)
        def _start_next():
            start(blk + 2, 0, sem0)

        flush(blk + 1, 1)
        return carry

    lax.fori_loop(0, _NBLK // 2, body, 0)


def kernel(idx, embedding):
    return _gather(idx.astype(jnp.int32), embedding)
